# Initial kernel scaffold; baseline (speedup 1.0000x reference)
import jax
import jax.numpy as jnp
from jax.experimental import pallas as pl


def kernel(inputs, embeddings):
    raise NotImplementedError("write your pallas kernel here")


# speedup vs baseline: 2.0129x; 2.0129x over previous
import functools

import jax
import jax.numpy as jnp
from jax import lax
from jax.experimental import pallas as pl
from jax.experimental.pallas import tpu as pltpu
from jax.experimental.pallas import tpu_sc as plsc

FEAT = 8
RES = 128
MAPS = 128
BATCH = 16384
BM = BATCH * MAPS
NC, NS = 2, 16
NW = NC * NS
PW = BM // NW
C = 1024
NCHUNK = PW // C
G = 4 * C // 128
NG = C // 16

_mesh = plsc.VectorSubcoreMesh(
    core_axis_name="c", subcore_axis_name="s", num_cores=NC, num_subcores=NS
)


@functools.partial(
    pl.kernel,
    out_type=jax.ShapeDtypeStruct((BM * 10,), jnp.float32),
    mesh=_mesh,
    scratch_types=[
        pltpu.VMEM((2 * C,), jnp.float32),
        pltpu.VMEM((G, 128), jnp.int32),
        pltpu.VMEM((4 * C,), jnp.float32),
        pltpu.VMEM((2 * C,), jnp.float32),
        pltpu.VMEM((G, 128, FEAT), jnp.float32),
        pltpu.VMEM((10 * C,), jnp.float32),
        pltpu.SemaphoreType.DMA,
    ],
    compiler_params=pltpu.CompilerParams(
        needs_layout_passes=False, use_tc_tiling_on_sc=False
    ),
)
def _dense_map_sc(in_hbm, emb_hbm, out_hbm, in_v, idx_v, w_v, xf_v, rows_v, out_v, sem):
    wid = lax.axis_index("s") * NC + lax.axis_index("c")
    base = wid * PW
    iota = lax.iota(jnp.int32, 16)

    def chunk_body(ci, carry):
        cbase = base + ci * C
        pltpu.sync_copy(in_hbm.at[pl.ds(cbase * 2, 2 * C)], in_v)

        def group_a(i, carry_a):
            p = i * 16 + iota
            xg = plsc.load_gather(in_v, [p * 2])
            yg = plsc.load_gather(in_v, [p * 2 + 1])
            m = (cbase + p) & (MAPS - 1)
            moff = m << 14
            xs = xg * jnp.float32(RES - 1)
            ys = yg * jnp.float32(RES - 1)
            xi = xs.astype(jnp.int32)
            yi = ys.astype(jnp.int32)
            xf = xs - xi.astype(jnp.float32)
            yf = ys - yi.astype(jnp.float32)
            id00 = moff + xi * RES + yi
            pr = p >> 7
            pc = p & 127
            wx0 = jnp.float32(1.0) - xf
            wy0 = jnp.float32(1.0) - yf
            ids = (id00, id00 + RES, id00 + 1, id00 + RES + 1)
            ws = (wx0 * wy0, xf * wy0, wx0 * yf, xf * yf)
            for k in range(4):
                plsc.store_scatter(idx_v, [pr + k * 8, pc], ids[k])
                plsc.store_scatter(w_v, [k * C + p], ws[k])
            plsc.store_scatter(xf_v, [p], xf)
            plsc.store_scatter(xf_v, [C + p], yf)
            return carry_a

        lax.fori_loop(0, NG, group_a, 0)

        copies = [
            pltpu.async_copy(emb_hbm.at[idx_v.at[d]], rows_v.at[d], sem)
            for d in range(G)
        ]
        for cp in copies:
            cp.wait()

        def group_b(i, carry_b):
            p = i * 16 + iota
            pr = p >> 7
            pc = p & 127
            wk = [plsc.load_gather(w_v, [k * C + p]) for k in range(4)]
            ob = p * 10
            for f in range(FEAT):
                fv = jnp.full((16,), f, jnp.int32)
                acc = wk[0] * plsc.load_gather(rows_v, [pr, pc, fv])
                for k in range(1, 4):
                    acc = acc + wk[k] * plsc.load_gather(rows_v, [pr + k * 8, pc, fv])
                plsc.store_scatter(out_v, [ob + f], acc)
            plsc.store_scatter(out_v, [ob + 8], plsc.load_gather(xf_v, [p]))
            plsc.store_scatter(out_v, [ob + 9], plsc.load_gather(xf_v, [C + p]))
            return carry_b

        lax.fori_loop(0, NG, group_b, 0)

        pltpu.sync_copy(out_v, out_hbm.at[pl.ds(cbase * 10, 10 * C)])
        return carry

    lax.fori_loop(0, NCHUNK, chunk_body, 0)


def kernel(inputs, embeddings):
    out = _dense_map_sc(inputs.reshape(BM * 2), embeddings)
    return out.reshape(BATCH, MAPS, 10)


# speedup vs baseline: 6.2889x; 3.1243x over previous
import functools

import jax
import jax.numpy as jnp
from jax import lax
from jax.experimental import pallas as pl
from jax.experimental.pallas import tpu as pltpu
from jax.experimental.pallas import tpu_sc as plsc

FEAT = 8
RES = 128
MAPS = 128
BATCH = 16384
BM = BATCH * MAPS
NC, NS = 2, 16
NW = NC * NS
PW = BM // NW
C = 1024
NCHUNK = PW // C
G = 4 * C // 128
NG = C // 16

_mesh = plsc.VectorSubcoreMesh(
    core_axis_name="c", subcore_axis_name="s", num_cores=NC, num_subcores=NS
)


@functools.partial(
    pl.kernel,
    out_type=jax.ShapeDtypeStruct((10 * BM,), jnp.float32),
    mesh=_mesh,
    scratch_types=[
        pltpu.VMEM((2 * C,), jnp.float32),
        pltpu.VMEM((G, 128), jnp.int32),
        pltpu.VMEM((4 * C,), jnp.float32),
        pltpu.VMEM((G, 128, FEAT), jnp.float32),
        pltpu.VMEM((10 * C,), jnp.float32),
        pltpu.SemaphoreType.DMA,
    ],
    compiler_params=pltpu.CompilerParams(
        needs_layout_passes=False, use_tc_tiling_on_sc=False
    ),
)
def _dense_map_sc(in_hbm, emb_hbm, out_hbm, in_v, idx_v, w_v, rows_v, out_v, sem):
    wid = lax.axis_index("s") * NC + lax.axis_index("c")
    base = wid * PW
    iota = lax.iota(jnp.int32, 16)

    def chunk_body(ci, carry):
        cbase = base + ci * C
        pltpu.sync_copy(in_hbm.at[pl.ds(cbase * 2, 2 * C)], in_v)

        def group_a(i, carry_a):
            p = i * 16 + iota
            pr = p >> 7
            pc = p & 127
            cbase_x = pr * 256 + pc
            xg = plsc.load_gather(in_v, [cbase_x])
            yg = plsc.load_gather(in_v, [cbase_x + 128])
            moff = pc << 14
            xs = xg * jnp.float32(RES - 1)
            ys = yg * jnp.float32(RES - 1)
            xi = xs.astype(jnp.int32)
            yi = ys.astype(jnp.int32)
            xf = xs - xi.astype(jnp.float32)
            yf = ys - yi.astype(jnp.float32)
            id00 = moff + xi * RES + yi
            wx0 = jnp.float32(1.0) - xf
            wy0 = jnp.float32(1.0) - yf
            ids = (id00, id00 + RES, id00 + 1, id00 + RES + 1)
            ws = (wx0 * wy0, xf * wy0, wx0 * yf, xf * yf)
            for k in range(4):
                plsc.store_scatter(idx_v, [pr + k * 8, pc], ids[k])
                plsc.store_scatter(w_v, [k * C + p], ws[k])
            out_v[pl.ds(8 * C + i * 16, 16)] = xf
            out_v[pl.ds(9 * C + i * 16, 16)] = yf
            return carry_a

        lax.fori_loop(0, NG, group_a, 0)

        copies = [
            pltpu.async_copy(emb_hbm.at[idx_v.at[d]], rows_v.at[d], sem)
            for d in range(G)
        ]
        for cp in copies:
            cp.wait()

        def group_b(i, carry_b):
            p = i * 16 + iota
            pr = p >> 7
            pc = p & 127
            wk = [plsc.load_gather(w_v, [k * C + p]) for k in range(4)]
            for f in range(FEAT):
                fv = jnp.full((16,), f, jnp.int32)
                acc = wk[0] * plsc.load_gather(rows_v, [pr, pc, fv])
                for k in range(1, 4):
                    acc = acc + wk[k] * plsc.load_gather(rows_v, [pr + k * 8, pc, fv])
                out_v[pl.ds(f * C + i * 16, 16)] = acc
            return carry_b

        lax.fori_loop(0, NG, group_b, 0)

        for f in range(10):
            pltpu.sync_copy(
                out_v.at[pl.ds(f * C, C)],
                out_hbm.at[pl.ds(f * BM + cbase, C)],
            )
        return carry

    lax.fori_loop(0, NCHUNK, chunk_body, 0)


def kernel(inputs, embeddings):
    coords = inputs.transpose(0, 2, 1).reshape(BM * 2)
    out = _dense_map_sc(coords, embeddings)
    return out.reshape(10, BATCH, MAPS).transpose(1, 2, 0)


# speedup vs baseline: 8.9083x; 1.4165x over previous
import functools

import jax
import jax.numpy as jnp
from jax import lax
from jax.experimental import pallas as pl
from jax.experimental.pallas import tpu as pltpu
from jax.experimental.pallas import tpu_sc as plsc

FEAT = 8
RES = 128
MAPS = 128
BATCH = 16384
BM = BATCH * MAPS
NC, NS = 2, 16
NW = NC * NS
PW = BM // NW
C = 1024
NCHUNK = PW // C
G = 4 * C // 128
NG = C // 16

_mesh = plsc.VectorSubcoreMesh(
    core_axis_name="c", subcore_axis_name="s", num_cores=NC, num_subcores=NS
)

NTILE = BM // 128
TPW = NTILE // NW
TB = 16
NBLK = TPW // TB


@functools.partial(
    pl.kernel,
    out_type=jax.ShapeDtypeStruct((BM * FEAT,), jnp.float32),
    mesh=_mesh,
    scratch_types=[
        pltpu.VMEM((2, TB, FEAT, 128), jnp.float32),
        pltpu.VMEM((2, TB * 1024), jnp.float32),
        pltpu.SemaphoreType.DMA,
        pltpu.SemaphoreType.DMA,
    ],
    compiler_params=pltpu.CompilerParams(
        needs_layout_passes=False, use_tc_tiling_on_sc=False
    ),
)
def _relayout_sc(emb_t, rows_hbm, in_v, out_v, sem_in, sem_out):
    wid = lax.axis_index("s") * NC + lax.axis_index("c")
    t0 = wid * TPW
    iota = lax.iota(jnp.int32, 16)

    def _in_args(b, par):
        return (emb_t.at[pl.ds(t0 + b * TB, TB), :, :], in_v.at[par], sem_in)

    def _out_args(b, par):
        return (
            out_v.at[par],
            rows_hbm.at[pl.ds((t0 + b * TB) * 1024, TB * 1024)],
            sem_out,
        )

    pltpu.async_copy(*_in_args(0, 0))
    pltpu.async_copy(*_in_args(1, 1))

    def blk2(bb, carry):
        for par in range(2):
            b = bb * 2 + par
            pltpu.make_async_copy(*_in_args(b, par)).wait()

            @pl.when(b >= 2)
            def _():
                pltpu.make_async_copy(*_out_args(b - 2, par)).wait()

            src = in_v.at[par]
            dst = out_v.at[par]

            def tile_body(tt, carry_t):
                ttv = jnp.full((16,), tt, jnp.int32)
                fv = iota & 7
                for i in range(64):
                    jv = i * 2 + (iota >> 3)
                    v = plsc.load_gather(src, [ttv, fv, jv])
                    dst[pl.ds(tt * 1024 + i * 16, 16)] = v
                return carry_t

            lax.fori_loop(0, TB, tile_body, 0)
            pltpu.async_copy(*_out_args(b, par))

            @pl.when(b + 2 < NBLK)
            def _():
                pltpu.async_copy(*_in_args(b + 2, par))
        return carry

    lax.fori_loop(0, NBLK // 2, blk2, 0)
    pltpu.make_async_copy(*_out_args(NBLK - 2, 0)).wait()
    pltpu.make_async_copy(*_out_args(NBLK - 1, 1)).wait()


@functools.partial(
    pl.kernel,
    out_type=jax.ShapeDtypeStruct((10 * BM,), jnp.float32),
    mesh=_mesh,
    scratch_types=[
        pltpu.VMEM((2 * C,), jnp.float32),
        pltpu.VMEM((G, 128), jnp.int32),
        pltpu.VMEM((4 * C,), jnp.float32),
        pltpu.VMEM((G, 128, FEAT), jnp.float32),
        pltpu.VMEM((10 * C,), jnp.float32),
        pltpu.SemaphoreType.DMA,
    ],
    compiler_params=pltpu.CompilerParams(
        needs_layout_passes=False, use_tc_tiling_on_sc=False
    ),
)
def _dense_map_sc(in_hbm, emb_hbm, out_hbm, in_v, idx_v, w_v, rows_v, out_v, sem):
    wid = lax.axis_index("s") * NC + lax.axis_index("c")
    base = wid * PW
    iota = lax.iota(jnp.int32, 16)

    def chunk_body(ci, carry):
        cbase = base + ci * C
        pltpu.sync_copy(in_hbm.at[pl.ds(cbase * 2, 2 * C)], in_v)

        def group_a(i, carry_a):
            p = i * 16 + iota
            pr = p >> 7
            pc = p & 127
            cbase_x = pr * 256 + pc
            xg = plsc.load_gather(in_v, [cbase_x])
            yg = plsc.load_gather(in_v, [cbase_x + 128])
            moff = pc << 14
            xs = xg * jnp.float32(RES - 1)
            ys = yg * jnp.float32(RES - 1)
            xi = xs.astype(jnp.int32)
            yi = ys.astype(jnp.int32)
            xf = xs - xi.astype(jnp.float32)
            yf = ys - yi.astype(jnp.float32)
            id00 = moff + xi * RES + yi
            wx0 = jnp.float32(1.0) - xf
            wy0 = jnp.float32(1.0) - yf
            ids = (id00, id00 + RES, id00 + 1, id00 + RES + 1)
            ws = (wx0 * wy0, xf * wy0, wx0 * yf, xf * yf)
            for k in range(4):
                plsc.store_scatter(idx_v, [pr + k * 8, pc], ids[k])
                plsc.store_scatter(w_v, [k * C + p], ws[k])
            out_v[pl.ds(8 * C + i * 16, 16)] = xf
            out_v[pl.ds(9 * C + i * 16, 16)] = yf
            return carry_a

        lax.fori_loop(0, NG, group_a, 0)

        copies = [
            pltpu.async_copy(emb_hbm.at[idx_v.at[d]], rows_v.at[d], sem)
            for d in range(G)
        ]
        for cp in copies:
            cp.wait()

        def group_b(i, carry_b):
            p = i * 16 + iota
            pr = p >> 7
            pc = p & 127
            wk = [plsc.load_gather(w_v, [k * C + p]) for k in range(4)]
            for f in range(FEAT):
                fv = jnp.full((16,), f, jnp.int32)
                acc = wk[0] * plsc.load_gather(rows_v, [pr, pc, fv])
                for k in range(1, 4):
                    acc = acc + wk[k] * plsc.load_gather(rows_v, [pr + k * 8, pc, fv])
                out_v[pl.ds(f * C + i * 16, 16)] = acc
            return carry_b

        lax.fori_loop(0, NG, group_b, 0)

        for f in range(10):
            pltpu.sync_copy(
                out_v.at[pl.ds(f * C, C)],
                out_hbm.at[pl.ds(f * BM + cbase, C)],
            )
        return carry

    lax.fori_loop(0, NCHUNK, chunk_body, 0)


def kernel(inputs, embeddings):
    coords = inputs.transpose(0, 2, 1).reshape(BM * 2)
    emb_t = embeddings.reshape(NTILE, 128, FEAT).transpose(0, 2, 1)
    rows = _relayout_sc(emb_t).reshape(BM, FEAT)
    out = _dense_map_sc(coords, rows)
    return out.reshape(10, BATCH, MAPS).transpose(1, 2, 0)


# speedup vs baseline: 13.0606x; 1.4661x over previous
import functools

import jax
import jax.numpy as jnp
from jax import lax
from jax.experimental import pallas as pl
from jax.experimental.pallas import tpu as pltpu
from jax.experimental.pallas import tpu_sc as plsc

FEAT = 8
RES = 128
MAPS = 128
BATCH = 16384
BM = BATCH * MAPS
NC, NS = 2, 16
NW = NC * NS
PW = BM // NW
C = 1024
NCHUNK = PW // C
G = 4 * C // 128
NG = C // 16

_mesh = plsc.VectorSubcoreMesh(
    core_axis_name="c", subcore_axis_name="s", num_cores=NC, num_subcores=NS
)

NTILE = BM // 128
TPW = NTILE // NW
TB = 16
NBLK = TPW // TB


@functools.partial(
    pl.kernel,
    out_type=jax.ShapeDtypeStruct((BM * FEAT,), jnp.float32),
    mesh=_mesh,
    scratch_types=[
        pltpu.VMEM((2, TB, FEAT, 128), jnp.float32),
        pltpu.VMEM((2, TB * 1024), jnp.float32),
        pltpu.SemaphoreType.DMA,
        pltpu.SemaphoreType.DMA,
    ],
    compiler_params=pltpu.CompilerParams(
        needs_layout_passes=False, use_tc_tiling_on_sc=False
    ),
)
def _relayout_sc(emb_t, rows_hbm, in_v, out_v, sem_in, sem_out):
    wid = lax.axis_index("s") * NC + lax.axis_index("c")
    t0 = wid * TPW
    iota = lax.iota(jnp.int32, 16)

    def _in_args(b, par):
        return (emb_t.at[pl.ds(t0 + b * TB, TB), :, :], in_v.at[par], sem_in)

    def _out_args(b, par):
        return (
            out_v.at[par],
            rows_hbm.at[pl.ds((t0 + b * TB) * 1024, TB * 1024)],
            sem_out,
        )

    pltpu.async_copy(*_in_args(0, 0))
    pltpu.async_copy(*_in_args(1, 1))

    def blk2(bb, carry):
        for par in range(2):
            b = bb * 2 + par
            pltpu.make_async_copy(*_in_args(b, par)).wait()

            @pl.when(b >= 2)
            def _():
                pltpu.make_async_copy(*_out_args(b - 2, par)).wait()

            src = in_v.at[par]
            dst = out_v.at[par]

            def tile_body(tt, carry_t):
                ttv = jnp.full((16,), tt, jnp.int32)
                fv = iota & 7
                for i in range(64):
                    jv = i * 2 + (iota >> 3)
                    v = plsc.load_gather(src, [ttv, fv, jv])
                    dst[pl.ds(tt * 1024 + i * 16, 16)] = v
                return carry_t

            lax.fori_loop(0, TB, tile_body, 0)
            pltpu.async_copy(*_out_args(b, par))

            @pl.when(b + 2 < NBLK)
            def _():
                pltpu.async_copy(*_in_args(b + 2, par))
        return carry

    lax.fori_loop(0, NBLK // 2, blk2, 0)
    pltpu.make_async_copy(*_out_args(NBLK - 2, 0)).wait()
    pltpu.make_async_copy(*_out_args(NBLK - 1, 1)).wait()


@functools.partial(
    pl.kernel,
    out_type=jax.ShapeDtypeStruct((10 * BM,), jnp.float32),
    mesh=_mesh,
    scratch_types=[
        pltpu.VMEM((2, 2 * C), jnp.float32),
        pltpu.VMEM((2, G, 128), jnp.int32),
        pltpu.VMEM((2, 4 * C), jnp.float32),
        pltpu.VMEM((2, G * 128, FEAT), jnp.float32),
        pltpu.VMEM((2, 10 * C), jnp.float32),
        pltpu.SemaphoreType.DMA,
        pltpu.SemaphoreType.DMA,
        pltpu.SemaphoreType.DMA,
    ],
    compiler_params=pltpu.CompilerParams(
        needs_layout_passes=False, use_tc_tiling_on_sc=False
    ),
)
def _dense_map_sc(
    in_hbm, emb_hbm, out_hbm, in_v, idx_v, w_v, rows_v, out_v,
    sem_in, sem_g, sem_out,
):
    wid = lax.axis_index("s") * NC + lax.axis_index("c")
    base = wid * PW
    iota = lax.iota(jnp.int32, 16)

    def _in_args(c, par):
        return (in_hbm.at[pl.ds((base + c * C) * 2, 2 * C)], in_v.at[par], sem_in)

    def _g_args(d, par):
        return (
            emb_hbm.at[idx_v.at[par, d]],
            rows_v.at[par, pl.ds(d * 128, 128), :],
            sem_g,
        )

    def _out_args(c, par, f):
        return (
            out_v.at[par, pl.ds(f * C, C)],
            out_hbm.at[pl.ds(f * BM + base + c * C, C)],
            sem_out,
        )

    def stage_a(c, par):
        def group_a(i, carry_a):
            p = i * 16 + iota
            pr = p >> 7
            pc = p & 127
            cbase_x = pr * 256 + pc
            xg = plsc.load_gather(in_v.at[par], [cbase_x])
            yg = plsc.load_gather(in_v.at[par], [cbase_x + 128])
            moff = pc << 14
            xs = xg * jnp.float32(RES - 1)
            ys = yg * jnp.float32(RES - 1)
            xi = xs.astype(jnp.int32)
            yi = ys.astype(jnp.int32)
            xf = xs - xi.astype(jnp.float32)
            yf = ys - yi.astype(jnp.float32)
            id00 = moff + xi * RES + yi
            wx0 = jnp.float32(1.0) - xf
            wy0 = jnp.float32(1.0) - yf
            ids = (id00, id00 + RES, id00 + 1, id00 + RES + 1)
            ws = (wx0 * wy0, xf * wy0, wx0 * yf, xf * yf)
            for k in range(4):
                plsc.store_scatter(idx_v.at[par], [pr + k * 8, pc], ids[k])
                plsc.store_scatter(w_v.at[par], [k * C + p], ws[k])
            out_v[par, pl.ds(8 * C + i * 16, 16)] = xf
            out_v[par, pl.ds(9 * C + i * 16, 16)] = yf
            return carry_a

        lax.fori_loop(0, NG, group_a, 0)

    def stage_b(c, par):
        def group_b(i, carry_b):
            p = i * 16 + iota
            wk = [plsc.load_gather(w_v.at[par], [k * C + p]) for k in range(4)]
            rk = [k * C + p for k in range(4)]
            for f in range(FEAT):
                fv = jnp.full((16,), f, jnp.int32)
                acc = wk[0] * plsc.load_gather(rows_v.at[par], [rk[0], fv])
                for k in range(1, 4):
                    acc = acc + wk[k] * plsc.load_gather(rows_v.at[par], [rk[k], fv])
                out_v[par, pl.ds(f * C + i * 16, 16)] = acc
            return carry_b

        lax.fori_loop(0, NG, group_b, 0)

    pltpu.async_copy(*_in_args(0, 0))
    pltpu.async_copy(*_in_args(1, 1))

    def pipe_body(c, carry):
        par = c & 1

        @pl.when(c >= 2)
        def _():
            for f in range(10):
                pltpu.make_async_copy(*_out_args(c - 2, par, f)).wait()

        @pl.when(c < NCHUNK)
        def _():
            pltpu.make_async_copy(*_in_args(c, par)).wait()
            stage_a(c, par)
            for d in range(G):
                pltpu.async_copy(*_g_args(d, par))

        @pl.when(c >= 1)
        def _():
            for d in range(G):
                pltpu.make_async_copy(*_g_args(d, 1 - par)).wait()
            stage_b(c - 1, 1 - par)
            for f in range(10):
                pltpu.async_copy(*_out_args(c - 1, 1 - par, f))

        @pl.when(c + 2 < NCHUNK)
        def _():
            pltpu.async_copy(*_in_args(c + 2, par))

        return carry

    lax.fori_loop(0, NCHUNK + 1, pipe_body, 0)
    for f in range(10):
        pltpu.make_async_copy(*_out_args(NCHUNK - 1, (NCHUNK - 1) & 1, f)).wait()


def kernel(inputs, embeddings):
    coords = inputs.transpose(0, 2, 1).reshape(BM * 2)
    emb_t = embeddings.reshape(NTILE, 128, FEAT).transpose(0, 2, 1)
    rows = _relayout_sc(emb_t).reshape(BM, FEAT)
    out = _dense_map_sc(coords, rows)
    return out.reshape(10, BATCH, MAPS).transpose(1, 2, 0)


# speedup vs baseline: 13.3023x; 1.0185x over previous
import functools

import jax
import jax.numpy as jnp
from jax import lax
from jax.experimental import pallas as pl
from jax.experimental.pallas import tpu as pltpu
from jax.experimental.pallas import tpu_sc as plsc

FEAT = 8
RES = 128
MAPS = 128
BATCH = 16384
BM = BATCH * MAPS
NC, NS = 2, 16
NW = NC * NS
PW = BM // NW
C = 1024
NCHUNK = PW // C
G = 4 * C // 128
NG = C // 16

_mesh = plsc.VectorSubcoreMesh(
    core_axis_name="c", subcore_axis_name="s", num_cores=NC, num_subcores=NS
)

NTILE = BM // 128
TPW = NTILE // NW
TB = 16
NBLK = TPW // TB


@functools.partial(
    pl.kernel,
    out_type=jax.ShapeDtypeStruct((BM * FEAT,), jnp.float32),
    mesh=_mesh,
    scratch_types=[
        pltpu.VMEM((2, TB * 1024), jnp.float32),
        pltpu.VMEM((2, TB * 1024), jnp.float32),
        pltpu.SemaphoreType.DMA,
        pltpu.SemaphoreType.DMA,
    ],
    compiler_params=pltpu.CompilerParams(
        needs_layout_passes=False, use_tc_tiling_on_sc=False
    ),
)
def _relayout_sc(emb_t, rows_hbm, in_v, out_v, sem_in, sem_out):
    wid = lax.axis_index("s") * NC + lax.axis_index("c")
    t0 = wid * TPW
    iota = lax.iota(jnp.int32, 16)
    bidx = (iota & 7) * 128 + (iota >> 3)

    def _in_args(b, par):
        return (
            emb_t.at[pl.ds((t0 + b * TB) * 1024, TB * 1024)],
            in_v.at[par],
            sem_in,
        )

    def _out_args(b, par):
        return (
            out_v.at[par],
            rows_hbm.at[pl.ds((t0 + b * TB) * 1024, TB * 1024)],
            sem_out,
        )

    pltpu.async_copy(*_in_args(0, 0))
    pltpu.async_copy(*_in_args(1, 1))

    def blk2(bb, carry):
        for par in range(2):
            b = bb * 2 + par
            pltpu.make_async_copy(*_in_args(b, par)).wait()

            @pl.when(b >= 2)
            def _():
                pltpu.make_async_copy(*_out_args(b - 2, par)).wait()

            src = in_v.at[par]

            def tile_body(tt, carry_t):
                tbase = tt * 1024
                for i in range(64):
                    v = plsc.load_gather(src, [bidx + (tbase + i * 2)])
                    out_v[par, pl.ds(tbase + i * 16, 16)] = v
                return carry_t

            lax.fori_loop(0, TB, tile_body, 0)
            pltpu.async_copy(*_out_args(b, par))

            @pl.when(b + 2 < NBLK)
            def _():
                pltpu.async_copy(*_in_args(b + 2, par))
        return carry

    lax.fori_loop(0, NBLK // 2, blk2, 0)
    pltpu.make_async_copy(*_out_args(NBLK - 2, 0)).wait()
    pltpu.make_async_copy(*_out_args(NBLK - 1, 1)).wait()


@functools.partial(
    pl.kernel,
    out_type=jax.ShapeDtypeStruct((10 * BM,), jnp.float32),
    mesh=_mesh,
    scratch_types=[
        pltpu.VMEM((2, 2 * C), jnp.float32),
        pltpu.VMEM((2, G, 128), jnp.int32),
        pltpu.VMEM((2, 4 * C), jnp.float32),
        pltpu.VMEM((2, G * 128, FEAT), jnp.float32),
        pltpu.VMEM((2, 10 * C), jnp.float32),
        pltpu.SemaphoreType.DMA,
        pltpu.SemaphoreType.DMA,
        pltpu.SemaphoreType.DMA,
    ],
    compiler_params=pltpu.CompilerParams(
        needs_layout_passes=False, use_tc_tiling_on_sc=False
    ),
)
def _dense_map_sc(
    in_hbm, emb_hbm, out_hbm, in_v, idx_v, w_v, rows_v, out_v,
    sem_in, sem_g, sem_out,
):
    wid = lax.axis_index("s") * NC + lax.axis_index("c")
    base = wid * PW
    iota = lax.iota(jnp.int32, 16)

    def _in_args(c, par):
        return (in_hbm.at[pl.ds((base + c * C) * 2, 2 * C)], in_v.at[par], sem_in)

    def _g_args(d, par):
        return (
            emb_hbm.at[idx_v.at[par, d]],
            rows_v.at[par, pl.ds(d * 128, 128), :],
            sem_g,
        )

    def _out_args(c, par, f):
        return (
            out_v.at[par, pl.ds(f * C, C)],
            out_hbm.at[pl.ds(f * BM + base + c * C, C)],
            sem_out,
        )

    iota8 = iota * 8

    def stage_a(c, par):
        def group_a(i, carry_a):
            br = i >> 3
            cs = (i & 7) * 16
            xg = in_v[par, pl.ds(br * 256 + cs, 16)]
            yg = in_v[par, pl.ds(br * 256 + 128 + cs, 16)]
            pc = cs + iota
            moff = pc << 14
            xs = xg * jnp.float32(RES - 1)
            ys = yg * jnp.float32(RES - 1)
            xi = xs.astype(jnp.int32)
            yi = ys.astype(jnp.int32)
            xf = xs - xi.astype(jnp.float32)
            yf = ys - yi.astype(jnp.float32)
            id00 = moff + xi * RES + yi
            wx0 = jnp.float32(1.0) - xf
            wy0 = jnp.float32(1.0) - yf
            ids = (id00, id00 + RES, id00 + 1, id00 + RES + 1)
            ws = (wx0 * wy0, xf * wy0, wx0 * yf, xf * yf)
            for k in range(4):
                idx_v[par, k * 8 + br, pl.ds(cs, 16)] = ids[k]
                w_v[par, pl.ds(k * C + i * 16, 16)] = ws[k]
            out_v[par, pl.ds(8 * C + i * 16, 16)] = xf
            out_v[par, pl.ds(9 * C + i * 16, 16)] = yf
            return carry_a

        lax.fori_loop(0, NG, group_a, 0)

    def stage_b(c, par):
        def group_b(i, carry_b):
            wk = [w_v[par, pl.ds(k * C + i * 16, 16)] for k in range(4)]
            p = i * 16 + iota
            rk = [p + k * C for k in range(4)]
            rows = rows_v.at[par]
            for f in range(FEAT):
                fv = jnp.full((16,), f, jnp.int32)
                acc = wk[0] * plsc.load_gather(rows, [rk[0], fv])
                for k in range(1, 4):
                    acc = acc + wk[k] * plsc.load_gather(rows, [rk[k], fv])
                out_v[par, pl.ds(f * C + i * 16, 16)] = acc
            return carry_b

        lax.fori_loop(0, NG, group_b, 0)

    pltpu.async_copy(*_in_args(0, 0))
    pltpu.async_copy(*_in_args(1, 1))

    def pipe_body(c, carry):
        par = c & 1

        @pl.when(c >= 2)
        def _():
            for f in range(10):
                pltpu.make_async_copy(*_out_args(c - 2, par, f)).wait()

        @pl.when(c < NCHUNK)
        def _():
            pltpu.make_async_copy(*_in_args(c, par)).wait()
            stage_a(c, par)
            for d in range(G):
                pltpu.async_copy(*_g_args(d, par))

        @pl.when(c >= 1)
        def _():
            for d in range(G):
                pltpu.make_async_copy(*_g_args(d, 1 - par)).wait()
            stage_b(c - 1, 1 - par)
            for f in range(10):
                pltpu.async_copy(*_out_args(c - 1, 1 - par, f))

        @pl.when(c + 2 < NCHUNK)
        def _():
            pltpu.async_copy(*_in_args(c + 2, par))

        return carry

    lax.fori_loop(0, NCHUNK + 1, pipe_body, 0)
    for f in range(10):
        pltpu.make_async_copy(*_out_args(NCHUNK - 1, (NCHUNK - 1) & 1, f)).wait()


def kernel(inputs, embeddings):
    coords = inputs.transpose(0, 2, 1).reshape(BM * 2)
    emb_t = embeddings.reshape(NTILE, 128, FEAT).transpose(0, 2, 1).reshape(NTILE * 1024)
    rows = _relayout_sc(emb_t).reshape(BM, FEAT)
    out = _dense_map_sc(coords, rows)
    return out.reshape(10, BATCH, MAPS).transpose(1, 2, 0)


# speedup vs baseline: 21.4446x; 1.6121x over previous
import functools

import jax
import jax.numpy as jnp
from jax import lax
from jax.experimental import pallas as pl
from jax.experimental.pallas import tpu as pltpu
from jax.experimental.pallas import tpu_sc as plsc

FEAT = 8
RES = 128
MAPS = 128
BATCH = 16384
BM = BATCH * MAPS
NC, NS = 2, 16
NW = NC * NS
PW = BM // NW
C = 1024
NCHUNK = PW // C
G = 4 * C // 128
NG = C // 16

_mesh = plsc.VectorSubcoreMesh(
    core_axis_name="c", subcore_axis_name="s", num_cores=NC, num_subcores=NS
)

NTILE = BM // 128
TPW = NTILE // NW
TB = 16
NBLK = TPW // TB


@functools.partial(
    pl.kernel,
    out_type=jax.ShapeDtypeStruct((BM * FEAT,), jnp.float32),
    mesh=_mesh,
    scratch_types=[
        pltpu.VMEM((2, TB * 1024), jnp.float32),
        pltpu.VMEM((2, TB * 1024), jnp.float32),
        pltpu.SemaphoreType.DMA,
        pltpu.SemaphoreType.DMA,
    ],
    compiler_params=pltpu.CompilerParams(
        needs_layout_passes=False, use_tc_tiling_on_sc=False
    ),
)
def _relayout_sc(emb_t, rows_hbm, in_v, out_v, sem_in, sem_out):
    wid = lax.axis_index("s") * NC + lax.axis_index("c")
    t0 = wid * TPW
    iota = lax.iota(jnp.int32, 16)
    bidx = (iota & 7) * 128 + (iota >> 3)

    def _in_args(b, par):
        return (
            emb_t.at[pl.ds((t0 + b * TB) * 1024, TB * 1024)],
            in_v.at[par],
            sem_in,
        )

    def _out_args(b, par):
        return (
            out_v.at[par],
            rows_hbm.at[pl.ds((t0 + b * TB) * 1024, TB * 1024)],
            sem_out,
        )

    pltpu.async_copy(*_in_args(0, 0))
    pltpu.async_copy(*_in_args(1, 1))

    def blk2(bb, carry):
        for par in range(2):
            b = bb * 2 + par
            pltpu.make_async_copy(*_in_args(b, par)).wait()

            @pl.when(b >= 2)
            def _():
                pltpu.make_async_copy(*_out_args(b - 2, par)).wait()

            src = in_v.at[par]

            @plsc.parallel_loop(0, TB)
            def tile_body(tt):
                tbase = tt * 1024
                for i in range(64):
                    v = plsc.load_gather(src, [bidx + (tbase + i * 2)])
                    out_v[par, pl.ds(tbase + i * 16, 16)] = v
            pltpu.async_copy(*_out_args(b, par))

            @pl.when(b + 2 < NBLK)
            def _():
                pltpu.async_copy(*_in_args(b + 2, par))
        return carry

    lax.fori_loop(0, NBLK // 2, blk2, 0)
    pltpu.make_async_copy(*_out_args(NBLK - 2, 0)).wait()
    pltpu.make_async_copy(*_out_args(NBLK - 1, 1)).wait()


@functools.partial(
    pl.kernel,
    out_type=jax.ShapeDtypeStruct((10 * BM,), jnp.float32),
    mesh=_mesh,
    scratch_types=[
        pltpu.VMEM((2, 2 * C), jnp.float32),
        pltpu.VMEM((2, G, 128), jnp.int32),
        pltpu.VMEM((2, 4 * C), jnp.float32),
        pltpu.VMEM((2, G * 128, FEAT), jnp.float32),
        pltpu.VMEM((2, 10 * C), jnp.float32),
        pltpu.SemaphoreType.DMA,
        pltpu.SemaphoreType.DMA,
        pltpu.SemaphoreType.DMA,
    ],
    compiler_params=pltpu.CompilerParams(
        needs_layout_passes=False, use_tc_tiling_on_sc=False
    ),
)
def _dense_map_sc(
    in_hbm, emb_hbm, out_hbm, in_v, idx_v, w_v, rows_v, out_v,
    sem_in, sem_g, sem_out,
):
    wid = lax.axis_index("s") * NC + lax.axis_index("c")
    base = wid * PW
    iota = lax.iota(jnp.int32, 16)

    def _in_args(c, par):
        return (in_hbm.at[pl.ds((base + c * C) * 2, 2 * C)], in_v.at[par], sem_in)

    def _g_args(d, par):
        return (
            emb_hbm.at[idx_v.at[par, d]],
            rows_v.at[par, pl.ds(d * 128, 128), :],
            sem_g,
        )

    def _out_args(c, par, f):
        return (
            out_v.at[par, pl.ds(f * C, C)],
            out_hbm.at[pl.ds(f * BM + base + c * C, C)],
            sem_out,
        )

    iota8 = iota * 8

    def stage_a(c, par):
        @plsc.parallel_loop(0, NG, unroll=2)
        def group_a(i):
            br = i >> 3
            cs = (i & 7) * 16
            xg = in_v[par, pl.ds(br * 256 + cs, 16)]
            yg = in_v[par, pl.ds(br * 256 + 128 + cs, 16)]
            pc = cs + iota
            moff = pc << 14
            xs = xg * jnp.float32(RES - 1)
            ys = yg * jnp.float32(RES - 1)
            xi = xs.astype(jnp.int32)
            yi = ys.astype(jnp.int32)
            xf = xs - xi.astype(jnp.float32)
            yf = ys - yi.astype(jnp.float32)
            id00 = moff + xi * RES + yi
            wx0 = jnp.float32(1.0) - xf
            wy0 = jnp.float32(1.0) - yf
            ids = (id00, id00 + RES, id00 + 1, id00 + RES + 1)
            ws = (wx0 * wy0, xf * wy0, wx0 * yf, xf * yf)
            for k in range(4):
                idx_v[par, k * 8 + br, pl.ds(cs, 16)] = ids[k]
                w_v[par, pl.ds(k * C + i * 16, 16)] = ws[k]
            out_v[par, pl.ds(8 * C + i * 16, 16)] = xf
            out_v[par, pl.ds(9 * C + i * 16, 16)] = yf

    def stage_b(c, par):
        @plsc.parallel_loop(0, NG, unroll=2)
        def group_b(i):
            wk = [w_v[par, pl.ds(k * C + i * 16, 16)] for k in range(4)]
            p = i * 16 + iota
            rk = [p + k * C for k in range(4)]
            rows = rows_v.at[par]
            for f0 in range(0, FEAT, 4):
                gs = []
                for f in range(f0, f0 + 4):
                    fv = jnp.full((16,), f, jnp.int32)
                    gs.append(
                        [plsc.load_gather(rows, [rk[k], fv]) for k in range(4)]
                    )
                for j, f in enumerate(range(f0, f0 + 4)):
                    g = gs[j]
                    acc = (wk[0] * g[0] + wk[1] * g[1]) + (
                        wk[2] * g[2] + wk[3] * g[3]
                    )
                    out_v[par, pl.ds(f * C + i * 16, 16)] = acc

    pltpu.async_copy(*_in_args(0, 0))
    pltpu.async_copy(*_in_args(1, 1))

    def pipe_body(c, carry):
        par = c & 1

        @pl.when(c >= 2)
        def _():
            for f in range(10):
                pltpu.make_async_copy(*_out_args(c - 2, par, f)).wait()

        @pl.when(c < NCHUNK)
        def _():
            pltpu.make_async_copy(*_in_args(c, par)).wait()
            stage_a(c, par)
            for d in range(G):
                pltpu.async_copy(*_g_args(d, par))

        @pl.when(c >= 1)
        def _():
            for d in range(G):
                pltpu.make_async_copy(*_g_args(d, 1 - par)).wait()
            stage_b(c - 1, 1 - par)
            for f in range(10):
                pltpu.async_copy(*_out_args(c - 1, 1 - par, f))

        @pl.when(c + 2 < NCHUNK)
        def _():
            pltpu.async_copy(*_in_args(c + 2, par))

        return carry

    lax.fori_loop(0, NCHUNK + 1, pipe_body, 0)
    for f in range(10):
        pltpu.make_async_copy(*_out_args(NCHUNK - 1, (NCHUNK - 1) & 1, f)).wait()


def kernel(inputs, embeddings):
    coords = inputs.transpose(0, 2, 1).reshape(BM * 2)
    emb_t = embeddings.reshape(NTILE, 128, FEAT).transpose(0, 2, 1).reshape(NTILE * 1024)
    rows = _relayout_sc(emb_t).reshape(BM, FEAT)
    out = _dense_map_sc(coords, rows)
    return out.reshape(10, BATCH, MAPS).transpose(1, 2, 0)


# speedup vs baseline: 22.8021x; 1.0633x over previous
import functools

import jax
import jax.numpy as jnp
from jax import lax
from jax.experimental import pallas as pl
from jax.experimental.pallas import tpu as pltpu
from jax.experimental.pallas import tpu_sc as plsc

FEAT = 8
RES = 128
MAPS = 128
BATCH = 16384
BM = BATCH * MAPS
NC, NS = 2, 16
NW = NC * NS
PW = BM // NW
C = 1024
NCHUNK = PW // C
G = 4 * C // 128
NG = C // 16

_mesh = plsc.VectorSubcoreMesh(
    core_axis_name="c", subcore_axis_name="s", num_cores=NC, num_subcores=NS
)

NTILE = BM // 128
TPW = NTILE // NW
TB = 16
NBLK = TPW // TB


@functools.partial(
    pl.kernel,
    out_type=jax.ShapeDtypeStruct((BM * FEAT,), jnp.float32),
    mesh=_mesh,
    scratch_types=[
        pltpu.VMEM((2, TB * 1024), jnp.float32),
        pltpu.VMEM((2, TB * 1024), jnp.float32),
        pltpu.SemaphoreType.DMA,
        pltpu.SemaphoreType.DMA,
    ],
    compiler_params=pltpu.CompilerParams(
        needs_layout_passes=False, use_tc_tiling_on_sc=False
    ),
)
def _relayout_sc(emb_t, rows_hbm, in_v, out_v, sem_in, sem_out):
    wid = lax.axis_index("s") * NC + lax.axis_index("c")
    t0 = wid * TPW
    iota = lax.iota(jnp.int32, 16)
    bidx = (iota & 7) * 128 + (iota >> 3)

    def _in_args(b, par):
        return (
            emb_t.at[pl.ds((t0 + b * TB) * 1024, TB * 1024)],
            in_v.at[par],
            sem_in,
        )

    def _out_args(b, par):
        return (
            out_v.at[par],
            rows_hbm.at[pl.ds((t0 + b * TB) * 1024, TB * 1024)],
            sem_out,
        )

    pltpu.async_copy(*_in_args(0, 0))
    pltpu.async_copy(*_in_args(1, 1))

    def blk2(bb, carry):
        for par in range(2):
            b = bb * 2 + par
            pltpu.make_async_copy(*_in_args(b, par)).wait()

            @pl.when(b >= 2)
            def _():
                pltpu.make_async_copy(*_out_args(b - 2, par)).wait()

            src = in_v.at[par]

            @plsc.parallel_loop(0, TB)
            def tile_body(tt):
                tbase = tt * 1024
                for i in range(64):
                    v = plsc.load_gather(src, [bidx + (tbase + i * 2)])
                    out_v[par, pl.ds(tbase + i * 16, 16)] = v
            pltpu.async_copy(*_out_args(b, par))

            @pl.when(b + 2 < NBLK)
            def _():
                pltpu.async_copy(*_in_args(b + 2, par))
        return carry

    lax.fori_loop(0, NBLK // 2, blk2, 0)
    pltpu.make_async_copy(*_out_args(NBLK - 2, 0)).wait()
    pltpu.make_async_copy(*_out_args(NBLK - 1, 1)).wait()


@functools.partial(
    pl.kernel,
    out_type=jax.ShapeDtypeStruct((10 * BM,), jnp.float32),
    mesh=_mesh,
    scratch_types=[
        pltpu.VMEM((2, 2 * C), jnp.float32),
        pltpu.VMEM((2, 4 * C), jnp.int32),
        pltpu.VMEM((2, 4 * C), jnp.float32),
        pltpu.VMEM((2, G * 128, FEAT), jnp.float32),
        pltpu.VMEM((2, 10 * C), jnp.float32),
        pltpu.SemaphoreType.DMA,
        pltpu.SemaphoreType.DMA,
        pltpu.SemaphoreType.DMA,
    ],
    compiler_params=pltpu.CompilerParams(
        needs_layout_passes=False, use_tc_tiling_on_sc=False
    ),
)
def _dense_map_sc(
    in_hbm, emb_hbm, out_hbm, in_v, idx_v, w_v, rows_v, out_v,
    sem_in, sem_g, sem_out,
):
    wid = lax.axis_index("s") * NC + lax.axis_index("c")
    base = wid * PW
    iota = lax.iota(jnp.int32, 16)

    def _in_args(c, par):
        return (in_hbm.at[pl.ds((base + c * C) * 2, 2 * C)], in_v.at[par], sem_in)

    def _g_args(d, par):
        return (
            emb_hbm.at[idx_v.at[par, pl.ds(d * 256, 256)]],
            rows_v.at[par, pl.ds(d * 256, 256), :],
            sem_g,
        )

    def _out_args(c, par, f):
        return (
            out_v.at[par, pl.ds(f * C, C)],
            out_hbm.at[pl.ds(f * BM + base + c * C, C)],
            sem_out,
        )

    iota8 = iota * 8

    def stage_a(c, par):
        @plsc.parallel_loop(0, NG, unroll=2)
        def group_a(i):
            br = i >> 3
            cs = (i & 7) * 16
            xg = in_v[par, pl.ds(br * 256 + cs, 16)]
            yg = in_v[par, pl.ds(br * 256 + 128 + cs, 16)]
            pc = cs + iota
            moff = pc << 14
            xs = xg * jnp.float32(RES - 1)
            ys = yg * jnp.float32(RES - 1)
            xi = xs.astype(jnp.int32)
            yi = ys.astype(jnp.int32)
            xf = xs - xi.astype(jnp.float32)
            yf = ys - yi.astype(jnp.float32)
            id00 = moff + xi * RES + yi
            wx0 = jnp.float32(1.0) - xf
            wy0 = jnp.float32(1.0) - yf
            ids = (id00, id00 + RES, id00 + 1, id00 + RES + 1)
            ws = (wx0 * wy0, xf * wy0, wx0 * yf, xf * yf)
            for k in range(4):
                idx_v[par, pl.ds(k * C + i * 16, 16)] = ids[k]
                w_v[par, pl.ds(k * C + i * 16, 16)] = ws[k]
            out_v[par, pl.ds(8 * C + i * 16, 16)] = xf
            out_v[par, pl.ds(9 * C + i * 16, 16)] = yf

    def stage_b(c, par):
        @plsc.parallel_loop(0, NG, unroll=4)
        def group_b(i):
            wk = [w_v[par, pl.ds(k * C + i * 16, 16)] for k in range(4)]
            p = i * 16 + iota
            rk = [p + k * C for k in range(4)]
            rows = rows_v.at[par]
            for f0 in range(0, FEAT, 4):
                gs = []
                for f in range(f0, f0 + 4):
                    fv = jnp.full((16,), f, jnp.int32)
                    gs.append(
                        [plsc.load_gather(rows, [rk[k], fv]) for k in range(4)]
                    )
                for j, f in enumerate(range(f0, f0 + 4)):
                    g = gs[j]
                    acc = (wk[0] * g[0] + wk[1] * g[1]) + (
                        wk[2] * g[2] + wk[3] * g[3]
                    )
                    out_v[par, pl.ds(f * C + i * 16, 16)] = acc

    pltpu.async_copy(*_in_args(0, 0))
    pltpu.async_copy(*_in_args(1, 1))

    def pipe_body(c, carry):
        par = c & 1

        @pl.when(c >= 2)
        def _():
            for f in range(10):
                pltpu.make_async_copy(*_out_args(c - 2, par, f)).wait()

        @pl.when(c < NCHUNK)
        def _():
            pltpu.make_async_copy(*_in_args(c, par)).wait()
            stage_a(c, par)
            for d in range(G // 2):
                pltpu.async_copy(*_g_args(d, par))

        @pl.when(c >= 1)
        def _():
            for d in range(G // 2):
                pltpu.make_async_copy(*_g_args(d, 1 - par)).wait()
            stage_b(c - 1, 1 - par)
            for f in range(10):
                pltpu.async_copy(*_out_args(c - 1, 1 - par, f))

        @pl.when(c + 2 < NCHUNK)
        def _():
            pltpu.async_copy(*_in_args(c + 2, par))

        return carry

    lax.fori_loop(0, NCHUNK + 1, pipe_body, 0)
    for f in range(10):
        pltpu.make_async_copy(*_out_args(NCHUNK - 1, (NCHUNK - 1) & 1, f)).wait()


def kernel(inputs, embeddings):
    coords = inputs.transpose(0, 2, 1).reshape(BM * 2)
    emb_t = embeddings.reshape(NTILE, 128, FEAT).transpose(0, 2, 1).reshape(NTILE * 1024)
    rows = _relayout_sc(emb_t).reshape(BM, FEAT)
    out = _dense_map_sc(coords, rows)
    return out.reshape(10, BATCH, MAPS).transpose(1, 2, 0)


# speedup vs baseline: 22.8501x; 1.0021x over previous
import functools

import jax
import jax.numpy as jnp
from jax import lax
from jax.experimental import pallas as pl
from jax.experimental.pallas import tpu as pltpu
from jax.experimental.pallas import tpu_sc as plsc

FEAT = 8
RES = 128
MAPS = 128
BATCH = 16384
BM = BATCH * MAPS
NC, NS = 2, 16
NW = NC * NS
PW = BM // NW
C = 1024
NCHUNK = PW // C
G = 4 * C // 128
NG = C // 16

_mesh = plsc.VectorSubcoreMesh(
    core_axis_name="c", subcore_axis_name="s", num_cores=NC, num_subcores=NS
)

NTILE = BM // 128
TPW = NTILE // NW
TB = 16
NBLK = TPW // TB


@functools.partial(
    pl.kernel,
    out_type=jax.ShapeDtypeStruct((BM * FEAT,), jnp.float32),
    mesh=_mesh,
    scratch_types=[
        pltpu.VMEM((2, TB * 1024), jnp.float32),
        pltpu.VMEM((2, TB * 1024), jnp.float32),
        pltpu.SemaphoreType.DMA,
        pltpu.SemaphoreType.DMA,
    ],
    compiler_params=pltpu.CompilerParams(
        needs_layout_passes=False, use_tc_tiling_on_sc=False
    ),
)
def _relayout_sc(emb_t, rows_hbm, in_v, out_v, sem_in, sem_out):
    wid = lax.axis_index("s") * NC + lax.axis_index("c")
    t0 = wid * TPW
    iota = lax.iota(jnp.int32, 16)
    bidx = (iota & 7) * 128 + (iota >> 3)

    def _in_args(b, par):
        return (
            emb_t.at[pl.ds((t0 + b * TB) * 1024, TB * 1024)],
            in_v.at[par],
            sem_in,
        )

    def _out_args(b, par):
        return (
            out_v.at[par],
            rows_hbm.at[pl.ds((t0 + b * TB) * 1024, TB * 1024)],
            sem_out,
        )

    pltpu.async_copy(*_in_args(0, 0))
    pltpu.async_copy(*_in_args(1, 1))

    def blk2(bb, carry):
        for par in range(2):
            b = bb * 2 + par
            pltpu.make_async_copy(*_in_args(b, par)).wait()

            @pl.when(b >= 2)
            def _():
                pltpu.make_async_copy(*_out_args(b - 2, par)).wait()

            src = in_v.at[par]

            @plsc.parallel_loop(0, TB)
            def tile_body(tt):
                tbase = tt * 1024
                for i in range(64):
                    v = plsc.load_gather(src, [bidx + (tbase + i * 2)])
                    out_v[par, pl.ds(tbase + i * 16, 16)] = v
            pltpu.async_copy(*_out_args(b, par))

            @pl.when(b + 2 < NBLK)
            def _():
                pltpu.async_copy(*_in_args(b + 2, par))
        return carry

    lax.fori_loop(0, NBLK // 2, blk2, 0)
    pltpu.make_async_copy(*_out_args(NBLK - 2, 0)).wait()
    pltpu.make_async_copy(*_out_args(NBLK - 1, 1)).wait()


@functools.partial(
    pl.kernel,
    out_type=jax.ShapeDtypeStruct((10 * BM,), jnp.float32),
    mesh=_mesh,
    scratch_types=[
        pltpu.VMEM((2, 2 * C), jnp.float32),
        pltpu.VMEM((2, 4 * C), jnp.int32),
        pltpu.VMEM((2, 4 * C), jnp.float32),
        pltpu.VMEM((2, G * 128, FEAT), jnp.float32),
        pltpu.VMEM((2, 10 * C), jnp.float32),
        pltpu.SemaphoreType.DMA,
        pltpu.SemaphoreType.DMA,
        pltpu.SemaphoreType.DMA,
    ],
    compiler_params=pltpu.CompilerParams(
        needs_layout_passes=False, use_tc_tiling_on_sc=False
    ),
)
def _dense_map_sc(
    in_hbm, emb_hbm, out_hbm, in_v, idx_v, w_v, rows_v, out_v,
    sem_in, sem_g, sem_out,
):
    wid = lax.axis_index("s") * NC + lax.axis_index("c")
    base = wid * PW
    iota = lax.iota(jnp.int32, 16)

    def _in_args(c, par):
        return (in_hbm.at[pl.ds((base + c * C) * 2, 2 * C)], in_v.at[par], sem_in)

    def _g_args(d, par):
        return (
            emb_hbm.at[idx_v.at[par, pl.ds(d * 512, 512)]],
            rows_v.at[par, pl.ds(d * 512, 512), :],
            sem_g,
        )

    def _out_args(c, par, f):
        return (
            out_v.at[par, pl.ds(f * C, C)],
            out_hbm.at[pl.ds(f * BM + base + c * C, C)],
            sem_out,
        )

    iota8 = iota * 8

    def stage_a(c, par):
        @plsc.parallel_loop(0, NG, unroll=4)
        def group_a(i):
            br = i >> 3
            cs = (i & 7) * 16
            xg = in_v[par, pl.ds(br * 256 + cs, 16)]
            yg = in_v[par, pl.ds(br * 256 + 128 + cs, 16)]
            pc = cs + iota
            moff = pc << 14
            xs = xg * jnp.float32(RES - 1)
            ys = yg * jnp.float32(RES - 1)
            xi = xs.astype(jnp.int32)
            yi = ys.astype(jnp.int32)
            xf = xs - xi.astype(jnp.float32)
            yf = ys - yi.astype(jnp.float32)
            id00 = moff + xi * RES + yi
            wx0 = jnp.float32(1.0) - xf
            wy0 = jnp.float32(1.0) - yf
            ids = (id00, id00 + RES, id00 + 1, id00 + RES + 1)
            ws = (wx0 * wy0, xf * wy0, wx0 * yf, xf * yf)
            for k in range(4):
                idx_v[par, pl.ds(k * C + i * 16, 16)] = ids[k]
                w_v[par, pl.ds(k * C + i * 16, 16)] = ws[k]
            out_v[par, pl.ds(8 * C + i * 16, 16)] = xf
            out_v[par, pl.ds(9 * C + i * 16, 16)] = yf

    def stage_b(c, par):
        @plsc.parallel_loop(0, NG, unroll=4)
        def group_b(i):
            wk = [w_v[par, pl.ds(k * C + i * 16, 16)] for k in range(4)]
            p = i * 16 + iota
            rk = [p + k * C for k in range(4)]
            rows = rows_v.at[par]
            for f0 in range(0, FEAT, 4):
                gs = []
                for f in range(f0, f0 + 4):
                    fv = jnp.full((16,), f, jnp.int32)
                    gs.append(
                        [plsc.load_gather(rows, [rk[k], fv]) for k in range(4)]
                    )
                for j, f in enumerate(range(f0, f0 + 4)):
                    g = gs[j]
                    acc = (wk[0] * g[0] + wk[1] * g[1]) + (
                        wk[2] * g[2] + wk[3] * g[3]
                    )
                    out_v[par, pl.ds(f * C + i * 16, 16)] = acc

    pltpu.async_copy(*_in_args(0, 0))
    pltpu.async_copy(*_in_args(1, 1))

    def pipe_body(c, carry):
        par = c & 1

        @pl.when(c >= 2)
        def _():
            for f in range(10):
                pltpu.make_async_copy(*_out_args(c - 2, par, f)).wait()

        @pl.when(c < NCHUNK)
        def _():
            pltpu.make_async_copy(*_in_args(c, par)).wait()
            stage_a(c, par)
            for d in range(G // 4):
                pltpu.async_copy(*_g_args(d, par))

        @pl.when(c >= 1)
        def _():
            for d in range(G // 4):
                pltpu.make_async_copy(*_g_args(d, 1 - par)).wait()
            stage_b(c - 1, 1 - par)
            for f in range(10):
                pltpu.async_copy(*_out_args(c - 1, 1 - par, f))

        @pl.when(c + 2 < NCHUNK)
        def _():
            pltpu.async_copy(*_in_args(c + 2, par))

        return carry

    lax.fori_loop(0, NCHUNK + 1, pipe_body, 0)
    for f in range(10):
        pltpu.make_async_copy(*_out_args(NCHUNK - 1, (NCHUNK - 1) & 1, f)).wait()


def kernel(inputs, embeddings):
    coords = inputs.transpose(0, 2, 1).reshape(BM * 2)
    emb_t = embeddings.reshape(NTILE, 128, FEAT).transpose(0, 2, 1).reshape(NTILE * 1024)
    rows = _relayout_sc(emb_t).reshape(BM, FEAT)
    out = _dense_map_sc(coords, rows)
    return out.reshape(10, BATCH, MAPS).transpose(1, 2, 0)
